# baseline (speedup 1.0000x reference)
import jax
import jax.numpy as jnp
from jax import lax
from jax.experimental import pallas as pl
from jax.experimental.pallas import tpu as pltpu
from jax.experimental.pallas import tpu_sc as plsc

B = 2
C = 32
G = 4
CG = C // G
H = 512
W = 512

BY = 16
WY = 40
NYB = H // BY
NTASK = B * NYB * G
NWORK = 32
TPW = NTASK // NWORK
NCH = WY // 8
PLANE = BY * W
EPS = 1e-7


def _floor_f32(x):
    t = x.astype(jnp.int32)
    tf = t.astype(jnp.float32)
    over = tf > x
    return jnp.where(over, t - 1, t), jnp.where(over, tf - 1.0, tf)


def _splat_body(frame_hbm, flow_hbm, imp_hbm, out_hbm,
                acc, src, flw, impw, fsem, psem, osem):
    nc = lax.axis_size("c")
    wid = lax.axis_index("s") * nc + lax.axis_index("c")
    lanes = lax.iota(jnp.int32, 16)
    zero16 = jnp.zeros((16,), jnp.float32)

    def task_body(i, _):
        strip_id = wid + NWORK * (i // G)
        b = strip_id // NYB
        y0 = (strip_id % NYB) * BY
        grp = lax.rem(i, G)
        first = grp == 0
        ys = pl.multiple_of(jnp.clip(y0 - 16, 0, H - WY), 8)
        lo = jnp.maximum(0, y0 - 9) - ys
        hi = jnp.minimum(H, y0 + BY + 8) - ys

        @plsc.parallel_loop(0, CG * PLANE // 16, unroll=4)
        def _(k):
            acc[pl.ds(k * 16, 16)] = zero16

        @pl.when(first)
        def _():
            @plsc.parallel_loop(0, PLANE // 16, unroll=4)
            def _(k):
                acc[pl.ds(CG * PLANE + k * 16, 16)] = zero16

        def issue_pf(ck):
            yr = pl.multiple_of(ys + ck * 8, 8)
            par = lax.rem(ck, 2)
            return pltpu.async_copy(
                flow_hbm.at[b, :, pl.ds(yr, 8), :], flw.at[par], psem)

        issue_pf(0)

        def chunk_body(ck, _):
            yr = pl.multiple_of(ys + ck * 8, 8)
            par = lax.rem(ck, 2)
            rlo = jnp.clip(lo - ck * 8, 0, 8)
            rhi = jnp.clip(hi - ck * 8, 0, 8)

            frame_src = frame_hbm.at[b, pl.ds(grp * CG, CG), pl.ds(yr, 8), :]

            @pl.when(rhi > rlo)
            def _():
                pltpu.async_copy(frame_src, src, fsem)

            pltpu.make_async_copy(
                flow_hbm.at[b, :, pl.ds(yr, 8), :], flw.at[par],
                psem).wait()

            @pl.when(ck + 1 < NCH)
            def _():
                issue_pf(ck + 1)

            @pl.when(rhi > rlo)
            def _():
                pltpu.sync_copy(imp_hbm.at[b, 0, pl.ds(yr, 8), :], impw)
                pltpu.make_async_copy(frame_src, src, fsem).wait()

                def row_body(rr, _):
                    gy = (yr + rr).astype(jnp.float32)

                    def mm_body(vc, mm):
                        fv = flw[par, 1, rr, pl.ds(vc * 16, 16)]
                        return (jnp.maximum(mm[0], fv),
                                jnp.minimum(mm[1], fv))

                    big = jnp.float32(1e30)
                    mx, mn = lax.fori_loop(
                        0, W // 16, mm_body,
                        (jnp.full((16,), -big), jnp.full((16,), big)))
                    fymax = jnp.max(mx)
                    fymin = jnp.min(mn)
                    hit = ((gy + fymax >= (y0 - 1).astype(jnp.float32))
                           & (gy + fymin < (y0 + BY).astype(jnp.float32)))

                    def vec(vc, with_imp):
                        xw = pl.ds(vc * 16, 16)
                        gx = (vc * 16 + lanes).astype(jnp.float32)
                        fltx = flw[par, 0, rr, xw] + gx
                        flty = flw[par, 1, rr, xw] + gy
                        ix0, x0f = _floor_f32(fltx)
                        iy0, y0f = _floor_f32(flty)
                        fx = fltx - x0f
                        fy = flty - y0f
                        wx0 = 1.0 - fx
                        wy0 = 1.0 - fy
                        ty0 = iy0 - y0
                        ty1 = ty0 + 1
                        my0 = plsc.bitcast(ty0, jnp.uint32) < BY
                        my1 = plsc.bitcast(ty1, jnp.uint32) < BY
                        mx0 = plsc.bitcast(ix0, jnp.uint32) < W
                        mx1 = plsc.bitcast(ix0 + 1, jnp.uint32) < W
                        m00 = mx0 & my0
                        m10 = mx1 & my0
                        m01 = mx0 & my1
                        m11 = mx1 & my1
                        impv = jnp.exp(impw[rr, xw])
                        w00 = wx0 * wy0 * impv
                        w10 = fx * wy0 * impv
                        w01 = wx0 * fy * impv
                        w11 = fx * fy * impv
                        i00 = ty0 * W + ix0
                        i10 = i00 + 1
                        i01 = i00 + W
                        i11 = i00 + W + 1
                        if with_imp:
                            aimp = acc.at[pl.ds(CG * PLANE, PLANE)]
                            plsc.addupdate_scatter(
                                aimp, [i00], w00, mask=m00)
                            plsc.addupdate_scatter(
                                aimp, [i10], w10, mask=m10)
                            plsc.addupdate_scatter(
                                aimp, [i01], w01, mask=m01)
                            plsc.addupdate_scatter(
                                aimp, [i11], w11, mask=m11)
                        for cl in range(CG):
                            fv = src[cl, rr, xw]
                            ac = acc.at[pl.ds(cl * PLANE, PLANE)]
                            plsc.addupdate_scatter(
                                ac, [i00], fv * w00, mask=m00)
                            plsc.addupdate_scatter(
                                ac, [i10], fv * w10, mask=m10)
                            plsc.addupdate_scatter(
                                ac, [i01], fv * w01, mask=m01)
                            plsc.addupdate_scatter(
                                ac, [i11], fv * w11, mask=m11)

                    @pl.when(hit & first)
                    def _():
                        @plsc.parallel_loop(0, W // 16, unroll=2)
                        def _(vc):
                            vec(vc, True)

                    @pl.when(hit & jnp.logical_not(first))
                    def _():
                        @plsc.parallel_loop(0, W // 16, unroll=2)
                        def _(vc):
                            vec(vc, False)

                    return 0

                lax.fori_loop(rlo, rhi, row_body, 0)

            return 0

        lax.fori_loop(0, NCH, chunk_body, 0)

        @pl.when(first)
        def _():
            @plsc.parallel_loop(0, PLANE // 16, unroll=4)
            def _(k):
                xw = pl.ds(CG * PLANE + k * 16, 16)
                acc[xw] = 1.0 / (acc[xw] + EPS)

        def norm_body(cl, _):
            @plsc.parallel_loop(0, PLANE // 16, unroll=4)
            def _(k):
                xw = pl.ds(cl * PLANE + k * 16, 16)
                rw = pl.ds(CG * PLANE + k * 16, 16)
                acc[xw] = acc[xw] * acc[rw]
            c = grp * CG + cl
            pltpu.async_copy(
                acc.at[pl.ds(cl * PLANE, PLANE)],
                out_hbm.at[pl.ds(((b * C + c) * H + y0) * W, PLANE)], osem)
            return 0

        lax.fori_loop(0, CG, norm_body, 0)

        def drain_body(cl, _):
            c = grp * CG + cl
            pltpu.make_async_copy(
                acc.at[pl.ds(cl * PLANE, PLANE)],
                out_hbm.at[pl.ds(((b * C + c) * H + y0) * W, PLANE)],
                osem).wait()
            return 0

        lax.fori_loop(0, CG, drain_body, 0)
        return 0

    lax.fori_loop(0, TPW, task_body, 0)


@jax.jit
def kernel(frame, flow, importance_metric):
    mesh = plsc.VectorSubcoreMesh(core_axis_name="c", subcore_axis_name="s")
    splat = pl.kernel(
        _splat_body,
        out_type=jax.ShapeDtypeStruct((B * C * H * W,), jnp.float32),
        mesh=mesh,
        compiler_params=pltpu.CompilerParams(
            use_tc_tiling_on_sc=False, needs_layout_passes=False),
        scratch_types=[
            pltpu.VMEM(((CG + 1) * PLANE,), jnp.float32),
            pltpu.VMEM((CG, 8, W), jnp.float32),
            pltpu.VMEM((2, 2, 8, W), jnp.float32),
            pltpu.VMEM((8, W), jnp.float32),
            pltpu.SemaphoreType.DMA,
            pltpu.SemaphoreType.DMA,
            pltpu.SemaphoreType.DMA,
        ],
    )
    return splat(frame, flow, importance_metric).reshape(B, C, H, W)


# speedup vs baseline: 1.2281x; 1.2281x over previous
import jax
import jax.numpy as jnp
from jax import lax
from jax.experimental import pallas as pl
from jax.experimental.pallas import tpu as pltpu
from jax.experimental.pallas import tpu_sc as plsc

B = 2
C = 32
G = 4
CG = C // G
H = 512
W = 512

BY = 16
WY = 40
NYB = H // BY
NTASK = B * NYB * G
NWORK = 32
TPW = NTASK // NWORK
NCH = WY // 8
PLANE = BY * W
EPS = 1e-7


def _floor_f32(x):
    t = x.astype(jnp.int32)
    tf = t.astype(jnp.float32)
    over = tf > x
    return jnp.where(over, t - 1, t), jnp.where(over, tf - 1.0, tf)


def _splat_body(frame_hbm, flow_hbm, imp_hbm, out_hbm,
                acc, src, flw, impw, fsem, psem, osem):
    nc = lax.axis_size("c")
    wid = lax.axis_index("s") * nc + lax.axis_index("c")
    lanes = lax.iota(jnp.int32, 16)
    zero16 = jnp.zeros((16,), jnp.float32)

    def task_body(i, _):
        strip_id = wid + NWORK * (i // G)
        b = strip_id // NYB
        y0 = (strip_id % NYB) * BY
        grp = lax.rem(i, G)
        ys = pl.multiple_of(jnp.clip(y0 - 16, 0, H - WY), 8)
        lo = jnp.maximum(0, y0 - 9) - ys
        hi = jnp.minimum(H, y0 + BY + 8) - ys

        @plsc.parallel_loop(0, (CG + 1) * PLANE // 16, unroll=4)
        def _(k):
            acc[pl.ds(k * 16, 16)] = zero16

        def issue_pf(ck):
            yr = pl.multiple_of(ys + ck * 8, 8)
            par = lax.rem(ck, 2)
            return pltpu.async_copy(
                flow_hbm.at[b, :, pl.ds(yr, 8), :], flw.at[par], psem)

        issue_pf(0)

        def chunk_body(ck, _):
            yr = pl.multiple_of(ys + ck * 8, 8)
            par = lax.rem(ck, 2)
            rlo = jnp.clip(lo - ck * 8, 0, 8)
            rhi = jnp.clip(hi - ck * 8, 0, 8)

            frame_src = frame_hbm.at[b, pl.ds(grp * CG, CG), pl.ds(yr, 8), :]

            @pl.when(rhi > rlo)
            def _():
                pltpu.async_copy(frame_src, src, fsem)

            pltpu.make_async_copy(
                flow_hbm.at[b, :, pl.ds(yr, 8), :], flw.at[par],
                psem).wait()

            @pl.when(ck + 1 < NCH)
            def _():
                issue_pf(ck + 1)

            @pl.when(rhi > rlo)
            def _():
                pltpu.sync_copy(imp_hbm.at[b, 0, pl.ds(yr, 8), :], impw)
                pltpu.make_async_copy(frame_src, src, fsem).wait()

                def row_body(rr, _):
                    gy = (yr + rr).astype(jnp.float32)

                    def mm_body(vc, mm):
                        fv = flw[par, 1, rr, pl.ds(vc * 16, 16)]
                        return (jnp.maximum(mm[0], fv),
                                jnp.minimum(mm[1], fv))

                    big = jnp.float32(1e30)
                    mx, mn = lax.fori_loop(
                        0, W // 16, mm_body,
                        (jnp.full((16,), -big), jnp.full((16,), big)))
                    fymax = jnp.max(mx)
                    fymin = jnp.min(mn)
                    hit = ((gy + fymax >= (y0 - 1).astype(jnp.float32))
                           & (gy + fymin < (y0 + BY).astype(jnp.float32)))

                    def vec(vc):
                        xw = pl.ds(vc * 16, 16)
                        gx = (vc * 16 + lanes).astype(jnp.float32)
                        fltx = flw[par, 0, rr, xw] + gx
                        flty = flw[par, 1, rr, xw] + gy
                        ix0, x0f = _floor_f32(fltx)
                        iy0, y0f = _floor_f32(flty)
                        fx = fltx - x0f
                        fy = flty - y0f
                        wx0 = 1.0 - fx
                        wy0 = 1.0 - fy
                        ty0 = iy0 - y0
                        ty1 = ty0 + 1
                        my0 = plsc.bitcast(ty0, jnp.uint32) < BY
                        my1 = plsc.bitcast(ty1, jnp.uint32) < BY
                        mx0 = plsc.bitcast(ix0, jnp.uint32) < W
                        mx1 = plsc.bitcast(ix0 + 1, jnp.uint32) < W
                        m00 = mx0 & my0
                        m10 = mx1 & my0
                        m01 = mx0 & my1
                        m11 = mx1 & my1
                        impv = jnp.exp(impw[rr, xw])
                        w00 = wx0 * wy0 * impv
                        w10 = fx * wy0 * impv
                        w01 = wx0 * fy * impv
                        w11 = fx * fy * impv
                        i00 = ty0 * W + ix0
                        i10 = i00 + 1
                        i01 = i00 + W
                        i11 = i00 + W + 1
                        aimp = acc.at[pl.ds(CG * PLANE, PLANE)]
                        plsc.addupdate_scatter(aimp, [i00], w00, mask=m00)
                        plsc.addupdate_scatter(aimp, [i10], w10, mask=m10)
                        plsc.addupdate_scatter(aimp, [i01], w01, mask=m01)
                        plsc.addupdate_scatter(aimp, [i11], w11, mask=m11)
                        for cl in range(CG):
                            fv = src[cl, rr, xw]
                            ac = acc.at[pl.ds(cl * PLANE, PLANE)]
                            plsc.addupdate_scatter(
                                ac, [i00], fv * w00, mask=m00)
                            plsc.addupdate_scatter(
                                ac, [i10], fv * w10, mask=m10)
                            plsc.addupdate_scatter(
                                ac, [i01], fv * w01, mask=m01)
                            plsc.addupdate_scatter(
                                ac, [i11], fv * w11, mask=m11)

                    @pl.when(hit)
                    def _():
                        @plsc.parallel_loop(0, W // 16, unroll=4)
                        def _(vc):
                            vec(vc)

                    return 0

                lax.fori_loop(rlo, rhi, row_body, 0)

            return 0

        lax.fori_loop(0, NCH, chunk_body, 0)

        @plsc.parallel_loop(0, PLANE // 16, unroll=4)
        def _(k):
            xw = pl.ds(CG * PLANE + k * 16, 16)
            acc[xw] = 1.0 / (acc[xw] + EPS)

        def norm_body(cl, _):
            @plsc.parallel_loop(0, PLANE // 16, unroll=4)
            def _(k):
                xw = pl.ds(cl * PLANE + k * 16, 16)
                rw = pl.ds(CG * PLANE + k * 16, 16)
                acc[xw] = acc[xw] * acc[rw]
            c = grp * CG + cl
            pltpu.async_copy(
                acc.at[pl.ds(cl * PLANE, PLANE)],
                out_hbm.at[pl.ds(((b * C + c) * H + y0) * W, PLANE)], osem)
            return 0

        lax.fori_loop(0, CG, norm_body, 0)

        def drain_body(cl, _):
            c = grp * CG + cl
            pltpu.make_async_copy(
                acc.at[pl.ds(cl * PLANE, PLANE)],
                out_hbm.at[pl.ds(((b * C + c) * H + y0) * W, PLANE)],
                osem).wait()
            return 0

        lax.fori_loop(0, CG, drain_body, 0)
        return 0

    lax.fori_loop(0, TPW, task_body, 0)


@jax.jit
def kernel(frame, flow, importance_metric):
    mesh = plsc.VectorSubcoreMesh(core_axis_name="c", subcore_axis_name="s")
    splat = pl.kernel(
        _splat_body,
        out_type=jax.ShapeDtypeStruct((B * C * H * W,), jnp.float32),
        mesh=mesh,
        compiler_params=pltpu.CompilerParams(
            use_tc_tiling_on_sc=False, needs_layout_passes=False),
        scratch_types=[
            pltpu.VMEM(((CG + 1) * PLANE,), jnp.float32),
            pltpu.VMEM((CG, 8, W), jnp.float32),
            pltpu.VMEM((2, 2, 8, W), jnp.float32),
            pltpu.VMEM((8, W), jnp.float32),
            pltpu.SemaphoreType.DMA,
            pltpu.SemaphoreType.DMA,
            pltpu.SemaphoreType.DMA,
        ],
    )
    return splat(frame, flow, importance_metric).reshape(B, C, H, W)


# speedup vs baseline: 1.2667x; 1.0314x over previous
import jax
import jax.numpy as jnp
from jax import lax
from jax.experimental import pallas as pl
from jax.experimental.pallas import tpu as pltpu
from jax.experimental.pallas import tpu_sc as plsc

B = 2
C = 32
G = 4
CG = C // G
H = 512
W = 512

BY = 16
WY = 40
NYB = H // BY
NTASK = B * NYB * G
NWORK = 32
TPW = NTASK // NWORK
NCH = WY // 8
PLANE = BY * W
EPS = 1e-7


def _floor_f32(x):
    t = x.astype(jnp.int32)
    tf = t.astype(jnp.float32)
    over = tf > x
    return jnp.where(over, t - 1, t), jnp.where(over, tf - 1.0, tf)


def _splat_body(frame_hbm, flow_hbm, imp_hbm, out_hbm,
                acc, src, flw, impw, fsem, psem, osem):
    nc = lax.axis_size("c")
    wid = lax.axis_index("s") * nc + lax.axis_index("c")
    lanes = lax.iota(jnp.int32, 16)
    zero16 = jnp.zeros((16,), jnp.float32)

    def task_body(i, _):
        strip_id = wid + NWORK * (i // G)
        b = strip_id // NYB
        y0 = (strip_id % NYB) * BY
        grp = lax.rem(i, G)
        ys = pl.multiple_of(jnp.clip(y0 - 16, 0, H - WY), 8)
        lo = jnp.maximum(0, y0 - 9) - ys
        hi = jnp.minimum(H, y0 + BY + 8) - ys

        @plsc.parallel_loop(0, (CG + 1) * PLANE // 16, unroll=4)
        def _(k):
            acc[pl.ds(k * 16, 16)] = zero16

        def issue_pf(ck):
            yr = pl.multiple_of(ys + ck * 8, 8)
            par = lax.rem(ck, 2)
            return pltpu.async_copy(
                flow_hbm.at[b, :, pl.ds(yr, 8), :], flw.at[par], psem)

        issue_pf(0)

        def chunk_body(ck, _):
            yr = pl.multiple_of(ys + ck * 8, 8)
            par = lax.rem(ck, 2)
            rlo = jnp.clip(lo - ck * 8, 0, 8)
            rhi = jnp.clip(hi - ck * 8, 0, 8)

            frame_src = frame_hbm.at[b, pl.ds(grp * CG, CG), pl.ds(yr, 8), :]

            @pl.when(rhi > rlo)
            def _():
                pltpu.async_copy(frame_src, src, fsem)

            pltpu.make_async_copy(
                flow_hbm.at[b, :, pl.ds(yr, 8), :], flw.at[par],
                psem).wait()

            @pl.when(ck + 1 < NCH)
            def _():
                issue_pf(ck + 1)

            @pl.when(rhi > rlo)
            def _():
                pltpu.sync_copy(imp_hbm.at[b, 0, pl.ds(yr, 8), :], impw)
                pltpu.make_async_copy(frame_src, src, fsem).wait()

                def row_body(rr, _):
                    gy = (yr + rr).astype(jnp.float32)

                    def mm_body(vc, mm):
                        fv = flw[par, 1, rr, pl.ds(vc * 16, 16)]
                        return (jnp.maximum(mm[0], fv),
                                jnp.minimum(mm[1], fv))

                    big = jnp.float32(1e30)
                    mx, mn = lax.fori_loop(
                        0, W // 16, mm_body,
                        (jnp.full((16,), -big), jnp.full((16,), big)))
                    fymax = jnp.max(mx)
                    fymin = jnp.min(mn)
                    hit = ((gy + fymax >= (y0 - 1).astype(jnp.float32))
                           & (gy + fymin < (y0 + BY).astype(jnp.float32)))

                    def vec(vc):
                        xw = pl.ds(vc * 16, 16)
                        gx = (vc * 16 + lanes).astype(jnp.float32)
                        fltx = flw[par, 0, rr, xw] + gx
                        flty = flw[par, 1, rr, xw] + gy
                        ix0, x0f = _floor_f32(fltx)
                        iy0, y0f = _floor_f32(flty)
                        fx = fltx - x0f
                        fy = flty - y0f
                        wx0 = 1.0 - fx
                        wy0 = 1.0 - fy
                        ty0 = iy0 - y0
                        ty1 = ty0 + 1
                        my0 = plsc.bitcast(ty0, jnp.uint32) < BY
                        my1 = plsc.bitcast(ty1, jnp.uint32) < BY
                        mx0 = plsc.bitcast(ix0, jnp.uint32) < W
                        mx1 = plsc.bitcast(ix0 + 1, jnp.uint32) < W
                        m00 = mx0 & my0
                        m10 = mx1 & my0
                        m01 = mx0 & my1
                        m11 = mx1 & my1
                        impv = jnp.exp(impw[rr, xw])
                        w00 = wx0 * wy0 * impv
                        w10 = fx * wy0 * impv
                        w01 = wx0 * fy * impv
                        w11 = fx * fy * impv
                        i00 = ty0 * W + ix0
                        i10 = i00 + 1
                        i01 = i00 + W
                        i11 = i00 + W + 1
                        aimp = acc.at[pl.ds(CG * PLANE, PLANE)]
                        plsc.addupdate_scatter(aimp, [i00], w00, mask=m00)
                        plsc.addupdate_scatter(aimp, [i10], w10, mask=m10)
                        plsc.addupdate_scatter(aimp, [i01], w01, mask=m01)
                        plsc.addupdate_scatter(aimp, [i11], w11, mask=m11)
                        for cl in range(CG):
                            fv = src[cl, rr, xw]
                            ac = acc.at[pl.ds(cl * PLANE, PLANE)]
                            plsc.addupdate_scatter(
                                ac, [i00], fv * w00, mask=m00)
                            plsc.addupdate_scatter(
                                ac, [i10], fv * w10, mask=m10)
                            plsc.addupdate_scatter(
                                ac, [i01], fv * w01, mask=m01)
                            plsc.addupdate_scatter(
                                ac, [i11], fv * w11, mask=m11)

                    @pl.when(hit)
                    def _():
                        @plsc.parallel_loop(0, W // 16, unroll=6)
                        def _(vc):
                            vec(vc)

                    return 0

                lax.fori_loop(rlo, rhi, row_body, 0)

            return 0

        lax.fori_loop(0, NCH, chunk_body, 0)

        @plsc.parallel_loop(0, PLANE // 16, unroll=4)
        def _(k):
            xw = pl.ds(CG * PLANE + k * 16, 16)
            acc[xw] = 1.0 / (acc[xw] + EPS)

        def norm_body(cl, _):
            @plsc.parallel_loop(0, PLANE // 16, unroll=4)
            def _(k):
                xw = pl.ds(cl * PLANE + k * 16, 16)
                rw = pl.ds(CG * PLANE + k * 16, 16)
                acc[xw] = acc[xw] * acc[rw]
            c = grp * CG + cl
            pltpu.async_copy(
                acc.at[pl.ds(cl * PLANE, PLANE)],
                out_hbm.at[pl.ds(((b * C + c) * H + y0) * W, PLANE)], osem)
            return 0

        lax.fori_loop(0, CG, norm_body, 0)

        def drain_body(cl, _):
            c = grp * CG + cl
            pltpu.make_async_copy(
                acc.at[pl.ds(cl * PLANE, PLANE)],
                out_hbm.at[pl.ds(((b * C + c) * H + y0) * W, PLANE)],
                osem).wait()
            return 0

        lax.fori_loop(0, CG, drain_body, 0)
        return 0

    lax.fori_loop(0, TPW, task_body, 0)


@jax.jit
def kernel(frame, flow, importance_metric):
    mesh = plsc.VectorSubcoreMesh(core_axis_name="c", subcore_axis_name="s")
    splat = pl.kernel(
        _splat_body,
        out_type=jax.ShapeDtypeStruct((B * C * H * W,), jnp.float32),
        mesh=mesh,
        compiler_params=pltpu.CompilerParams(
            use_tc_tiling_on_sc=False, needs_layout_passes=False),
        scratch_types=[
            pltpu.VMEM(((CG + 1) * PLANE,), jnp.float32),
            pltpu.VMEM((CG, 8, W), jnp.float32),
            pltpu.VMEM((2, 2, 8, W), jnp.float32),
            pltpu.VMEM((8, W), jnp.float32),
            pltpu.SemaphoreType.DMA,
            pltpu.SemaphoreType.DMA,
            pltpu.SemaphoreType.DMA,
        ],
    )
    return splat(frame, flow, importance_metric).reshape(B, C, H, W)
